# baseline (speedup 1.0000x reference)
import functools

import jax
import jax.numpy as jnp
from jax import lax
from jax.experimental import pallas as pl
from jax.experimental.pallas import tpu as pltpu
from jax.experimental.pallas import tpu_sc as plsc

_L = 16
_RES = 13
_CAP = 800
_NCHUNK = 25
_LABCHUNK = 1024
_CAP2 = 1088


def _emb_kernel(tt_hbm, idx_hbm, out_hbm, stage, labv, hita, hitq,
                hitl0, hitl1, hitp0, hitp1, posb0, posb1,
                rowb, sem_st, sem_lab, sem_sc0, sem_sc1, *, num_cores,
                batch, hidden, base_cols, extra_cols):
    w = lax.axis_index("s") * num_cores + lax.axis_index("c")
    c0 = base_cols * w + jnp.minimum(w, extra_cols)
    c1 = c0 + base_cols + jnp.where(w < extra_cols, 1, 0)
    iota = lax.iota(jnp.int32, _L)
    trash = jnp.full((_L,), batch, jnp.int32)
    hitl = (hitl0, hitl1)
    hitp = (hitp0, hitp1)
    posb = (posb0, posb1)

    def fire(r):
        for i in range(_RES):
            col = c0 + _RES * r + i

            @pl.when(col < c1)
            def _():
                pltpu.async_copy(
                    tt_hbm.at[:, pl.ds(col * 128, 128)],
                    stage.at[pl.ds(i * 64, 64)], sem_st)

    def drain(r):
        for i in range(_RES):
            col = c0 + _RES * r + i

            @pl.when(col < c1)
            def _():
                pltpu.make_async_copy(
                    tt_hbm.at[:, pl.ds(0, 128)],
                    stage.at[pl.ds(i * 64, 64)], sem_st).wait()

    fire(0)

    pad0 = jnp.broadcast_to((c0 * 128).astype(jnp.int32), (_L,))
    pad1 = jnp.broadcast_to(((c0 + _RES) * 128).astype(jnp.int32), (_L,))
    for g in range(_CAP // _L):
        hitl0[pl.ds(g * _L, _L)] = pad0
        hitl1[pl.ds(g * _L, _L)] = pad1
        hitp0[pl.ds(g * _L, _L)] = trash
        hitp1[pl.ds(g * _L, _L)] = trash
    for g in range(_CAP2 // _L):
        hita[pl.ds(g * _L, _L)] = pad0
        hitq[pl.ds(g * _L, _L)] = trash

    nchunks = batch // _LABCHUNK
    pltpu.async_copy(idx_hbm.at[pl.ds(0, _LABCHUNK)],
                     labv.at[pl.ds(0, _LABCHUNK)], sem_lab)

    def scan_chunk(ch, nm):
        pltpu.make_async_copy(idx_hbm.at[pl.ds(0, _LABCHUNK)],
                              labv.at[pl.ds(0, _LABCHUNK)], sem_lab).wait()

        @pl.when(ch + 1 < nchunks)
        def _():
            pltpu.async_copy(
                idx_hbm.at[pl.ds((ch + 1) * _LABCHUNK, _LABCHUNK)],
                labv.at[pl.ds(((ch + 1) & 1) * _LABCHUNK, _LABCHUNK)],
                sem_lab)

        half = (ch & 1) * _LABCHUNK

        def scan_vec(v, nm):
            base = half + v * 4 * _L
            pbase = ch * _LABCHUNK + v * 4 * _L
            labs = [labv[pl.ds(base + k * _L, _L)] for k in range(4)]
            cols = [lax.shift_right_logical(x, 7) for x in labs]
            ms = [(c >= c0) & (c < c1) for c in cols]
            cnts = [plsc.all_reduce_population_count(m) for m in ms]
            for k in range(4):
                pos = pbase + k * _L + iota
                plsc.store_compressed(hita.at[pl.ds(nm, _L)], labs[k],
                                      mask=ms[k])
                plsc.store_compressed(hitq.at[pl.ds(nm, _L)], pos,
                                      mask=ms[k])
                nm = nm + cnts[k][0]
            return nm

        return lax.fori_loop(0, _LABCHUNK // (4 * _L), scan_vec, nm)

    nm = lax.fori_loop(0, nchunks, scan_chunk, jnp.int32(0))

    def split_vec(u, carry):
        n0, n1 = carry
        va = hita[pl.ds(u * _L, _L)]
        vq = hitq[pl.ds(u * _L, _L)]
        colv = lax.shift_right_logical(va, 7)
        mr1 = colv >= c0 + _RES
        mr0 = jnp.logical_not(mr1)
        c0n = plsc.all_reduce_population_count(mr0)
        c1n = plsc.all_reduce_population_count(mr1)
        plsc.store_compressed(hitl0.at[pl.ds(n0, _L)], va, mask=mr0)
        plsc.store_compressed(hitp0.at[pl.ds(n0, _L)], vq, mask=mr0)
        plsc.store_compressed(hitl1.at[pl.ds(n1, _L)], va, mask=mr1)
        plsc.store_compressed(hitp1.at[pl.ds(n1, _L)], vq, mask=mr1)
        return n0 + c0n[0], n1 + c1n[0]

    nsplit = lax.div(nm + (_L - 1), jnp.int32(_L))
    n0, n1 = lax.fori_loop(0, nsplit, split_vec,
                           (jnp.int32(0), jnp.int32(0)))

    for r in range(2):
        for k in range(_NCHUNK):
            for j in range(2):
                posb[r].at[k][pl.ds(j * _L, _L)] = (
                    hitp[r][pl.ds(k * 32 + j * _L, _L)])

    def issue_scatter(src, idx_row, parity):
        @pl.when(parity == 0)
        def _():
            pltpu.async_copy(src, out_hbm.at[idx_row], sem_sc0)

        @pl.when(parity == 1)
        def _():
            pltpu.async_copy(src, out_hbm.at[idx_row], sem_sc1)

    def wait_scatter(parity):
        @pl.when(parity == 0)
        def _():
            pltpu.make_async_copy(
                rowb.at[pl.ds(0, 32)], out_hbm.at[posb0.at[0]],
                sem_sc0).wait()

        @pl.when(parity == 1)
        def _():
            pltpu.make_async_copy(
                rowb.at[pl.ds(0, 32)], out_hbm.at[posb0.at[0]],
                sem_sc1).wait()

    def extract_round(r, nh, counts_in):
        drain(r)
        ngroups = lax.div(nh + (_L - 1), jnp.int32(_L))

        def g_body(g, counts):
            niss0, nw0, niss1, nw1 = counts
            parity = (g >> 1) & 1
            pend0 = (parity == 0) & (niss0 > nw0)
            pend1 = (parity == 1) & (niss1 > nw1)

            @pl.when(((g & 1) == 0) & pend0)
            def _():
                wait_scatter(jnp.int32(0))

            @pl.when(((g & 1) == 0) & pend1)
            def _():
                wait_scatter(jnp.int32(1))

            chunk_start = (g & 1) == 0
            nw0 = nw0 + jnp.where(chunk_start & pend0, 1, 0)
            nw1 = nw1 + jnp.where(chunk_start & pend1, 1, 0)

            lvec = hitl[r][pl.ds(g * _L, _L)]
            slotbase = (g & 3) * _L
            for lane in range(_L):
                l = lvec[lane]
                cl = lax.shift_right_logical(l, 7) - (c0 + _RES * r)
                mm = l & 127
                for j in range(4):
                    ridx = cl * 64 + j * _L + iota
                    cidx = jnp.broadcast_to(mm, (_L,))
                    vals = plsc.load_gather(stage, [ridx, cidx])
                    rowb.at[slotbase + lane][pl.ds(j * _L, _L)] = vals

            @pl.when((g & 1) == 1)
            def _():
                issue_scatter(rowb.at[pl.ds(parity * 32, 32)],
                              posb[r].at[g >> 1], parity)

            last = (g & 1) == 1
            niss0 = niss0 + jnp.where(last & (parity == 0), 1, 0)
            niss1 = niss1 + jnp.where(last & (parity == 1), 1, 0)
            return niss0, nw0, niss1, nw1

        counts = lax.fori_loop(0, ngroups, g_body, counts_in)
        niss0, nw0, niss1, nw1 = counts

        tail = (ngroups & 1) != 0
        tparity = (ngroups >> 1) & 1

        @pl.when(tail)
        def _():
            issue_scatter(rowb.at[pl.ds(tparity * 32, 32)],
                          posb[r].at[ngroups >> 1], tparity)

        niss0 = niss0 + jnp.where(tail & (tparity == 0), 1, 0)
        niss1 = niss1 + jnp.where(tail & (tparity == 1), 1, 0)
        return niss0, nw0, niss1, nw1

    counts = extract_round(0, n0, (jnp.int32(0),) * 4)
    fire(1)
    niss0, nw0, niss1, nw1 = extract_round(1, n1, counts)

    def drain0(i, carry):
        wait_scatter(jnp.int32(0))
        return carry

    def drain1(i, carry):
        wait_scatter(jnp.int32(1))
        return carry

    lax.fori_loop(0, niss0 - nw0, drain0, jnp.int32(0))
    lax.fori_loop(0, niss1 - nw1, drain1, jnp.int32(0))


def kernel(labels, embedding_table):
    (batch,) = labels.shape
    rows, hidden = embedding_table.shape
    info = plsc.get_sparse_core_info()
    num_workers = info.num_cores * info.num_subcores
    cols = -(-rows // 128)
    base_cols = cols // num_workers
    extra_cols = cols % num_workers

    tt = embedding_table.T

    mesh = plsc.VectorSubcoreMesh(core_axis_name="c", subcore_axis_name="s")

    emb = pl.kernel(
        functools.partial(
            _emb_kernel,
            num_cores=info.num_cores,
            batch=batch,
            hidden=hidden,
            base_cols=base_cols,
            extra_cols=extra_cols,
        ),
        out_type=jax.ShapeDtypeStruct((batch + 8, 128), jnp.float32),
        mesh=mesh,
        scratch_types=[
            pltpu.VMEM((_RES * 64, 128), jnp.float32),
            pltpu.VMEM((2 * _LABCHUNK,), jnp.int32),
            pltpu.VMEM((_CAP2,), jnp.int32),
            pltpu.VMEM((_CAP2,), jnp.int32),
            pltpu.VMEM((_CAP,), jnp.int32),
            pltpu.VMEM((_CAP,), jnp.int32),
            pltpu.VMEM((_CAP,), jnp.int32),
            pltpu.VMEM((_CAP,), jnp.int32),
            pltpu.VMEM((_NCHUNK, 32), jnp.int32),
            pltpu.VMEM((_NCHUNK, 32), jnp.int32),
            pltpu.VMEM((64, 128), jnp.float32),
            pltpu.SemaphoreType.DMA,
            pltpu.SemaphoreType.DMA,
            pltpu.SemaphoreType.DMA,
            pltpu.SemaphoreType.DMA,
        ],
        compiler_params=pltpu.CompilerParams(
            use_tc_tiling_on_sc=True, needs_layout_passes=False),
    )
    out = emb(tt, labels.astype(jnp.int32))
    return out[:batch, :hidden]


# speedup vs baseline: 1.0240x; 1.0240x over previous
import functools

import jax
import jax.numpy as jnp
from jax import lax
from jax.experimental import pallas as pl
from jax.experimental.pallas import tpu as pltpu
from jax.experimental.pallas import tpu_sc as plsc

_L = 16
_RES = 13
_CAP = 800
_NCHUNK = 25
_LABCHUNK = 1024
_CAP2 = 1088


def _emb_kernel(tt_hbm, idx_hbm, out_hbm, stage, labv, hita, hitq,
                hitl0, hitl1, hitp0, hitp1, posb0, posb1,
                rowb, sem_st, sem_lab, sem_sc0, sem_sc1, *, num_cores,
                batch, hidden, base_cols, extra_cols):
    w = lax.axis_index("s") * num_cores + lax.axis_index("c")
    c0 = base_cols * w + jnp.minimum(w, extra_cols)
    c1 = c0 + base_cols + jnp.where(w < extra_cols, 1, 0)
    iota = lax.iota(jnp.int32, _L)
    trash = jnp.full((_L,), batch, jnp.int32)
    hitl = (hitl0, hitl1)
    hitp = (hitp0, hitp1)
    posb = (posb0, posb1)

    def fire(r):
        for i in range(_RES):
            col = c0 + _RES * r + i

            @pl.when(col < c1)
            def _():
                pltpu.async_copy(
                    tt_hbm.at[:, pl.ds(col * 128, 128)],
                    stage.at[pl.ds(i * 64, 64)], sem_st)

    def drain(r):
        for i in range(_RES):
            col = c0 + _RES * r + i

            @pl.when(col < c1)
            def _():
                pltpu.make_async_copy(
                    tt_hbm.at[:, pl.ds(0, 128)],
                    stage.at[pl.ds(i * 64, 64)], sem_st).wait()

    fire(0)

    pad0 = jnp.broadcast_to((c0 * 128).astype(jnp.int32), (_L,))
    pad1 = jnp.broadcast_to(((c0 + _RES) * 128).astype(jnp.int32), (_L,))
    for g in range(_CAP // _L):
        hitl0[pl.ds(g * _L, _L)] = pad0
        hitl1[pl.ds(g * _L, _L)] = pad1
        hitp0[pl.ds(g * _L, _L)] = trash
        hitp1[pl.ds(g * _L, _L)] = trash
    for g in range(_CAP2 // _L):
        hita[pl.ds(g * _L, _L)] = pad0
        hitq[pl.ds(g * _L, _L)] = trash

    nchunks = batch // _LABCHUNK
    pltpu.async_copy(idx_hbm.at[pl.ds(0, _LABCHUNK)],
                     labv.at[pl.ds(0, _LABCHUNK)], sem_lab)

    def scan_chunk(ch, nm):
        pltpu.make_async_copy(idx_hbm.at[pl.ds(0, _LABCHUNK)],
                              labv.at[pl.ds(0, _LABCHUNK)], sem_lab).wait()

        @pl.when(ch + 1 < nchunks)
        def _():
            pltpu.async_copy(
                idx_hbm.at[pl.ds((ch + 1) * _LABCHUNK, _LABCHUNK)],
                labv.at[pl.ds(((ch + 1) & 1) * _LABCHUNK, _LABCHUNK)],
                sem_lab)

        half = (ch & 1) * _LABCHUNK

        def scan_vec(v, nm):
            base = half + v * 4 * _L
            pbase = ch * _LABCHUNK + v * 4 * _L
            labs = [labv[pl.ds(base + k * _L, _L)] for k in range(4)]
            cols = [lax.shift_right_logical(x, 7) for x in labs]
            ms = [(c >= c0) & (c < c1) for c in cols]
            cnts = [plsc.all_reduce_population_count(m) for m in ms]
            for k in range(4):
                pos = pbase + k * _L + iota
                plsc.store_compressed(hita.at[pl.ds(nm, _L)], labs[k],
                                      mask=ms[k])
                plsc.store_compressed(hitq.at[pl.ds(nm, _L)], pos,
                                      mask=ms[k])
                nm = nm + cnts[k][0]
            return nm

        return lax.fori_loop(0, _LABCHUNK // (4 * _L), scan_vec, nm)

    nm = lax.fori_loop(0, nchunks, scan_chunk, jnp.int32(0))

    def split_vec(u, carry):
        n0, n1 = carry
        va = hita[pl.ds(u * _L, _L)]
        vq = hitq[pl.ds(u * _L, _L)]
        colv = lax.shift_right_logical(va, 7)
        mr1 = colv >= c0 + _RES
        mr0 = jnp.logical_not(mr1)
        c0n = plsc.all_reduce_population_count(mr0)
        c1n = plsc.all_reduce_population_count(mr1)
        plsc.store_compressed(hitl0.at[pl.ds(n0, _L)], va, mask=mr0)
        plsc.store_compressed(hitp0.at[pl.ds(n0, _L)], vq, mask=mr0)
        plsc.store_compressed(hitl1.at[pl.ds(n1, _L)], va, mask=mr1)
        plsc.store_compressed(hitp1.at[pl.ds(n1, _L)], vq, mask=mr1)
        return n0 + c0n[0], n1 + c1n[0]

    nsplit = lax.div(nm + (_L - 1), jnp.int32(_L))
    n0, n1 = lax.fori_loop(0, nsplit, split_vec,
                           (jnp.int32(0), jnp.int32(0)))

    for r in range(2):
        for k in range(_NCHUNK):
            for j in range(2):
                posb[r].at[k][pl.ds(j * _L, _L)] = (
                    hitp[r][pl.ds(k * 32 + j * _L, _L)])

    def issue_scatter(src, idx_row, parity):
        @pl.when(parity == 0)
        def _():
            pltpu.async_copy(src, out_hbm.at[idx_row], sem_sc0)

        @pl.when(parity == 1)
        def _():
            pltpu.async_copy(src, out_hbm.at[idx_row], sem_sc1)

    def wait_scatter(parity):
        @pl.when(parity == 0)
        def _():
            pltpu.make_async_copy(
                rowb.at[pl.ds(0, 32)], out_hbm.at[posb0.at[0]],
                sem_sc0).wait()

        @pl.when(parity == 1)
        def _():
            pltpu.make_async_copy(
                rowb.at[pl.ds(0, 32)], out_hbm.at[posb0.at[0]],
                sem_sc1).wait()

    def extract_round(r, nh, counts_in):
        drain(r)
        ngroups = lax.div(nh + (_L - 1), jnp.int32(_L))

        def g_body(g, counts):
            niss0, nw0, niss1, nw1 = counts
            parity = (g >> 1) & 1
            pend0 = (parity == 0) & (niss0 > nw0)
            pend1 = (parity == 1) & (niss1 > nw1)

            @pl.when(((g & 1) == 0) & pend0)
            def _():
                wait_scatter(jnp.int32(0))

            @pl.when(((g & 1) == 0) & pend1)
            def _():
                wait_scatter(jnp.int32(1))

            chunk_start = (g & 1) == 0
            nw0 = nw0 + jnp.where(chunk_start & pend0, 1, 0)
            nw1 = nw1 + jnp.where(chunk_start & pend1, 1, 0)

            lvec = hitl[r][pl.ds(g * _L, _L)]
            slotvec = (g & 3) * _L + iota
            clv = lax.shift_right_logical(lvec, 7) - (c0 + _RES * r)
            mv = lvec & 127
            rowidx = clv * 64
            for d in range(64):
                vals = plsc.load_gather(stage, [rowidx + d, mv])
                plsc.store_scatter(rowb, [slotvec, jnp.full((_L,), d,
                                                           jnp.int32)],
                                   vals)

            @pl.when((g & 1) == 1)
            def _():
                issue_scatter(rowb.at[pl.ds(parity * 32, 32)],
                              posb[r].at[g >> 1], parity)

            last = (g & 1) == 1
            niss0 = niss0 + jnp.where(last & (parity == 0), 1, 0)
            niss1 = niss1 + jnp.where(last & (parity == 1), 1, 0)
            return niss0, nw0, niss1, nw1

        counts = lax.fori_loop(0, ngroups, g_body, counts_in)
        niss0, nw0, niss1, nw1 = counts

        tail = (ngroups & 1) != 0
        tparity = (ngroups >> 1) & 1

        @pl.when(tail)
        def _():
            issue_scatter(rowb.at[pl.ds(tparity * 32, 32)],
                          posb[r].at[ngroups >> 1], tparity)

        niss0 = niss0 + jnp.where(tail & (tparity == 0), 1, 0)
        niss1 = niss1 + jnp.where(tail & (tparity == 1), 1, 0)
        return niss0, nw0, niss1, nw1

    counts = extract_round(0, n0, (jnp.int32(0),) * 4)
    fire(1)
    niss0, nw0, niss1, nw1 = extract_round(1, n1, counts)

    def drain0(i, carry):
        wait_scatter(jnp.int32(0))
        return carry

    def drain1(i, carry):
        wait_scatter(jnp.int32(1))
        return carry

    lax.fori_loop(0, niss0 - nw0, drain0, jnp.int32(0))
    lax.fori_loop(0, niss1 - nw1, drain1, jnp.int32(0))


def kernel(labels, embedding_table):
    (batch,) = labels.shape
    rows, hidden = embedding_table.shape
    info = plsc.get_sparse_core_info()
    num_workers = info.num_cores * info.num_subcores
    cols = -(-rows // 128)
    base_cols = cols // num_workers
    extra_cols = cols % num_workers

    tt = embedding_table.T

    mesh = plsc.VectorSubcoreMesh(core_axis_name="c", subcore_axis_name="s")

    emb = pl.kernel(
        functools.partial(
            _emb_kernel,
            num_cores=info.num_cores,
            batch=batch,
            hidden=hidden,
            base_cols=base_cols,
            extra_cols=extra_cols,
        ),
        out_type=jax.ShapeDtypeStruct((batch + 8, 128), jnp.float32),
        mesh=mesh,
        scratch_types=[
            pltpu.VMEM((_RES * 64, 128), jnp.float32),
            pltpu.VMEM((2 * _LABCHUNK,), jnp.int32),
            pltpu.VMEM((_CAP2,), jnp.int32),
            pltpu.VMEM((_CAP2,), jnp.int32),
            pltpu.VMEM((_CAP,), jnp.int32),
            pltpu.VMEM((_CAP,), jnp.int32),
            pltpu.VMEM((_CAP,), jnp.int32),
            pltpu.VMEM((_CAP,), jnp.int32),
            pltpu.VMEM((_NCHUNK, 32), jnp.int32),
            pltpu.VMEM((_NCHUNK, 32), jnp.int32),
            pltpu.VMEM((64, 128), jnp.float32),
            pltpu.SemaphoreType.DMA,
            pltpu.SemaphoreType.DMA,
            pltpu.SemaphoreType.DMA,
            pltpu.SemaphoreType.DMA,
        ],
        compiler_params=pltpu.CompilerParams(
            use_tc_tiling_on_sc=True, needs_layout_passes=False),
    )
    out = emb(tt, labels.astype(jnp.int32))
    return out[:batch, :hidden]


# speedup vs baseline: 1.1495x; 1.1226x over previous
import functools

import jax
import jax.numpy as jnp
from jax import lax
from jax.experimental import pallas as pl
from jax.experimental.pallas import tpu as pltpu
from jax.experimental.pallas import tpu_sc as plsc

_L = 16
_RES = 13
_CAP = 512
_NCHUNK = 32
_LABCHUNK = 512
_CAP2 = 960


def _emb_kernel(tt_hbm, idx_hbm, out_hbm, stage, labv, hita, hitq,
                hitl0, hitl1, hitp0, hitp1, posb0, posb1,
                rowb, sem_st, sem_lab, sem_sc0, sem_sc1, sem_sc2, sem_sc3,
                *, num_cores, batch, hidden, base_cols, extra_cols):
    w = lax.axis_index("s") * num_cores + lax.axis_index("c")
    c0 = base_cols * w + jnp.minimum(w, extra_cols)
    c1 = c0 + base_cols + jnp.where(w < extra_cols, 1, 0)
    iota = lax.iota(jnp.int32, _L)
    trash = jnp.full((_L,), batch, jnp.int32)
    hitl = (hitl0, hitl1)
    hitp = (hitp0, hitp1)
    posb = (posb0, posb1)

    def fire(r):
        for i in range(_RES):
            col = c0 + _RES * r + i

            @pl.when(col < c1)
            def _():
                pltpu.async_copy(
                    tt_hbm.at[:, pl.ds(col * 128, 128)],
                    stage.at[pl.ds(i * 64, 64)], sem_st)

    def drain(r):
        for i in range(_RES):
            col = c0 + _RES * r + i

            @pl.when(col < c1)
            def _():
                pltpu.make_async_copy(
                    tt_hbm.at[:, pl.ds(0, 128)],
                    stage.at[pl.ds(i * 64, 64)], sem_st).wait()

    fire(0)

    pad0 = jnp.broadcast_to((c0 * 128).astype(jnp.int32), (_L,))
    pad1 = jnp.broadcast_to(((c0 + _RES) * 128).astype(jnp.int32), (_L,))
    for g in range(_CAP // _L):
        hitl0[pl.ds(g * _L, _L)] = pad0
        hitl1[pl.ds(g * _L, _L)] = pad1
        hitp0[pl.ds(g * _L, _L)] = trash
        hitp1[pl.ds(g * _L, _L)] = trash
    for g in range(_CAP2 // _L):
        hita[pl.ds(g * _L, _L)] = pad0
        hitq[pl.ds(g * _L, _L)] = trash

    nchunks = batch // _LABCHUNK
    pltpu.async_copy(idx_hbm.at[pl.ds(0, _LABCHUNK)],
                     labv.at[pl.ds(0, _LABCHUNK)], sem_lab)

    def scan_chunk(ch, nm):
        pltpu.make_async_copy(idx_hbm.at[pl.ds(0, _LABCHUNK)],
                              labv.at[pl.ds(0, _LABCHUNK)], sem_lab).wait()

        @pl.when(ch + 1 < nchunks)
        def _():
            pltpu.async_copy(
                idx_hbm.at[pl.ds((ch + 1) * _LABCHUNK, _LABCHUNK)],
                labv.at[pl.ds(((ch + 1) & 1) * _LABCHUNK, _LABCHUNK)],
                sem_lab)

        half = (ch & 1) * _LABCHUNK

        def scan_vec(v, nm):
            base = half + v * 4 * _L
            pbase = ch * _LABCHUNK + v * 4 * _L
            labs = [labv[pl.ds(base + k * _L, _L)] for k in range(4)]
            cols = [lax.shift_right_logical(x, 7) for x in labs]
            ms = [(c >= c0) & (c < c1) for c in cols]
            cnts = [plsc.all_reduce_population_count(m) for m in ms]
            for k in range(4):
                pos = pbase + k * _L + iota
                plsc.store_compressed(hita.at[pl.ds(nm, _L)], labs[k],
                                      mask=ms[k])
                plsc.store_compressed(hitq.at[pl.ds(nm, _L)], pos,
                                      mask=ms[k])
                nm = nm + cnts[k][0]
            return nm

        return lax.fori_loop(0, _LABCHUNK // (4 * _L), scan_vec, nm)

    with jax.named_scope("scan"):
        nm = lax.fori_loop(0, nchunks, scan_chunk, jnp.int32(0))

    def split_vec(u, carry):
        n0, n1 = carry
        va = hita[pl.ds(u * _L, _L)]
        vq = hitq[pl.ds(u * _L, _L)]
        colv = lax.shift_right_logical(va, 7)
        mr1 = colv >= c0 + _RES
        mr0 = jnp.logical_not(mr1)
        c0n = plsc.all_reduce_population_count(mr0)
        c1n = plsc.all_reduce_population_count(mr1)
        plsc.store_compressed(hitl0.at[pl.ds(n0, _L)], va, mask=mr0)
        plsc.store_compressed(hitp0.at[pl.ds(n0, _L)], vq, mask=mr0)
        plsc.store_compressed(hitl1.at[pl.ds(n1, _L)], va, mask=mr1)
        plsc.store_compressed(hitp1.at[pl.ds(n1, _L)], vq, mask=mr1)
        return n0 + c0n[0], n1 + c1n[0]

    with jax.named_scope("split"):
        nsplit = lax.div(nm + (_L - 1), jnp.int32(_L))
        n0, n1 = lax.fori_loop(0, nsplit, split_vec,
                               (jnp.int32(0), jnp.int32(0)))

    for r in range(2):
        for k in range(_NCHUNK):
            posb[r].at[k][pl.ds(0, _L)] = hitp[r][pl.ds(k * _L, _L)]

    sems = (sem_sc0, sem_sc1, sem_sc2, sem_sc3)

    def issue_scatter(src_rows, idx_row, parity):
        for p in range(4):
            @pl.when(parity == p)
            def _():
                pltpu.async_copy(src_rows, out_hbm.at[idx_row], sems[p])

    def wait_scatter_p(p):
        pltpu.make_async_copy(
            rowb.at[pl.ds(0, 16)], out_hbm.at[posb0.at[0]],
            sems[p]).wait()

    def extract_round(r, nh, pends_in):
        drain(r)
        ngroups = lax.div(nh + (_L - 1), jnp.int32(_L))

        def g_body(g, pends):
            parity = g & 3
            new_pends = []
            for p in range(4):
                hit = (parity == p) & (pends[p] > 0)

                @pl.when(hit)
                def _():
                    wait_scatter_p(p)

                new_pends.append(jnp.where(hit, 0, pends[p]))
            pends = new_pends

            lvec = hitl[r][pl.ds(g * _L, _L)]
            slotvec = (g & 3) * _L + iota
            clv = lax.shift_right_logical(lvec, 7) - (c0 + _RES * r)
            mv = lvec & 127
            rowidx = clv * 64
            for d in range(64):
                vals = plsc.load_gather(stage, [rowidx + d, mv])
                plsc.store_scatter(rowb, [slotvec, jnp.full((_L,), d,
                                                           jnp.int32)],
                                   vals)

            issue_scatter(rowb.at[pl.ds(parity * _L, _L)],
                          posb[r].at[g], parity)
            pends = [jnp.where(parity == p, 1, pends[p])
                     for p in range(4)]
            return tuple(pends)

        return lax.fori_loop(0, ngroups, g_body, pends_in)

    with jax.named_scope("extract0"):
        pends = extract_round(0, n0, (jnp.int32(0),) * 4)
    with jax.named_scope("stage1"):
        fire(1)
    with jax.named_scope("extract1"):
        pends = extract_round(1, n1, pends)

    for p in range(4):
        @pl.when(pends[p] > 0)
        def _():
            wait_scatter_p(p)


def kernel(labels, embedding_table):
    (batch,) = labels.shape
    rows, hidden = embedding_table.shape
    info = plsc.get_sparse_core_info()
    num_workers = info.num_cores * info.num_subcores
    cols = -(-rows // 128)
    base_cols = cols // num_workers
    extra_cols = cols % num_workers

    tt = embedding_table.T

    mesh = plsc.VectorSubcoreMesh(core_axis_name="c", subcore_axis_name="s")

    emb = pl.kernel(
        functools.partial(
            _emb_kernel,
            num_cores=info.num_cores,
            batch=batch,
            hidden=hidden,
            base_cols=base_cols,
            extra_cols=extra_cols,
        ),
        out_type=jax.ShapeDtypeStruct((batch + 8, 128), jnp.float32),
        mesh=mesh,
        scratch_types=[
            pltpu.VMEM((_RES * 64, 128), jnp.float32),
            pltpu.VMEM((2 * _LABCHUNK,), jnp.int32),
            pltpu.VMEM((_CAP2,), jnp.int32),
            pltpu.VMEM((_CAP2,), jnp.int32),
            pltpu.VMEM((_CAP,), jnp.int32),
            pltpu.VMEM((_CAP,), jnp.int32),
            pltpu.VMEM((_CAP,), jnp.int32),
            pltpu.VMEM((_CAP,), jnp.int32),
            pltpu.VMEM((_NCHUNK, _L), jnp.int32),
            pltpu.VMEM((_NCHUNK, _L), jnp.int32),
            pltpu.VMEM((64, 128), jnp.float32),
            pltpu.SemaphoreType.DMA,
            pltpu.SemaphoreType.DMA,
            pltpu.SemaphoreType.DMA,
            pltpu.SemaphoreType.DMA,
            pltpu.SemaphoreType.DMA,
            pltpu.SemaphoreType.DMA,
        ],
        compiler_params=pltpu.CompilerParams(
            use_tc_tiling_on_sc=True, needs_layout_passes=False),
    )
    out = emb(tt, labels.astype(jnp.int32))
    return out[:batch, :hidden]


# speedup vs baseline: 1.3377x; 1.1637x over previous
import functools

import jax
import jax.numpy as jnp
from jax import lax
from jax.experimental import pallas as pl
from jax.experimental.pallas import tpu as pltpu
from jax.experimental.pallas import tpu_sc as plsc

_CHUNK = 128


def _emb_kernel(table_hbm, idx_hbm, out_hbm, idx_v, rows_v, sem, *,
                num_cores, rows_per_worker, hidden):
    wid = lax.axis_index("s") * num_cores + lax.axis_index("c")
    base = wid * rows_per_worker
    pltpu.sync_copy(idx_hbm.at[pl.ds(base, rows_per_worker)], idx_v)
    copies = [
        pltpu.async_copy(
            table_hbm.at[idx_v.at[pl.ds(j * _CHUNK, _CHUNK)]],
            rows_v.at[pl.ds(j * _CHUNK, _CHUNK)],
            sem,
        )
        for j in range(rows_per_worker // _CHUNK)
    ]
    for c in copies:
        c.wait()
    pltpu.sync_copy(rows_v, out_hbm.at[pl.ds(base, rows_per_worker)])


def kernel(labels, embedding_table):
    (batch,) = labels.shape
    rows, hidden = embedding_table.shape
    info = plsc.get_sparse_core_info()
    num_workers = info.num_cores * info.num_subcores
    rows_per_worker = batch // num_workers

    rpad = (-rows) % 8
    tpad = jnp.pad(embedding_table, ((0, rpad), (0, 128 - hidden)))

    mesh = plsc.VectorSubcoreMesh(core_axis_name="c", subcore_axis_name="s")

    emb = pl.kernel(
        functools.partial(
            _emb_kernel,
            num_cores=info.num_cores,
            rows_per_worker=rows_per_worker,
            hidden=hidden,
        ),
        out_type=jax.ShapeDtypeStruct((batch, 128), jnp.float32),
        mesh=mesh,
        scratch_types=[
            pltpu.VMEM((rows_per_worker,), jnp.int32),
            pltpu.VMEM((rows_per_worker, 128), jnp.float32),
            pltpu.SemaphoreType.DMA,
        ],
        compiler_params=pltpu.CompilerParams(use_tc_tiling_on_sc=True),
    )
    return emb(tpad, labels.astype(jnp.int32))[:, :hidden]
